# Initial kernel scaffold; baseline (speedup 1.0000x reference)
import jax
import jax.numpy as jnp
from jax.experimental import pallas as pl


def kernel(x, edge_attr, edge_index, bases, atom_emb, bond_emb, fW1, fb1, fs1, ft1, fW2, fb2, fs2, ft2, pW, pb, W1, b1, s1, t1, W2, b2, s2, t2, Wp, bp):
    raise NotImplementedError("write your pallas kernel here")


# speedup vs baseline: 2.8609x; 2.8609x over previous
import functools

import jax
import jax.numpy as jnp
from jax import lax
from jax.experimental import pallas as pl
from jax.experimental.pallas import tpu as pltpu
from jax.experimental.pallas import tpu_sc as plsc

N = 10000
E = 320000
H = 128
L = 4

NC = 2
NS = 16
NTILE = NC * NS
EPT = E // NTILE
CH = 80
NCHUNK = EPT // CH
NP = 10112
RPT = NP // NS

BE = 3200
BN = 2000

_f32 = jnp.float32


def _filt_body(bases_ref, ea_ref, w1_ref, c1_ref, w2_ref, c2_ref,
               expb_ref, comb_ref):
    b = jnp.dot(bases_ref[...], w1_ref[...], preferred_element_type=_f32)
    b = jax.nn.gelu(b + c1_ref[...], approximate=True)
    b = jnp.dot(b, w2_ref[...], preferred_element_type=_f32)
    b = jax.nn.gelu(b + c2_ref[...], approximate=True)
    expb_ref[...] = jnp.exp(b)
    ea = ea_ref[...]
    comb_ref[...] = ea[:, 0:1] * 25 + ea[:, 1:2] * 5 + ea[:, 2:3]


def _filt_call(bases, edge_attr, w1e, c1e, w2e, c2e):
    return pl.pallas_call(
        _filt_body,
        grid=(E // BE,),
        in_specs=[
            pl.BlockSpec((BE, 16), lambda i: (i, 0)),
            pl.BlockSpec((BE, 3), lambda i: (i, 0)),
            pl.BlockSpec((16, H), lambda i: (0, 0)),
            pl.BlockSpec((1, H), lambda i: (0, 0)),
            pl.BlockSpec((H, H), lambda i: (0, 0)),
            pl.BlockSpec((1, H), lambda i: (0, 0)),
        ],
        out_specs=[
            pl.BlockSpec((BE, H), lambda i: (i, 0)),
            pl.BlockSpec((BE, 1), lambda i: (i, 0)),
        ],
        out_shape=[
            jax.ShapeDtypeStruct((E, H), _f32),
            jax.ShapeDtypeStruct((E, 1), jnp.int32),
        ],
    )(bases, edge_attr, w1e, c1e, w2e, c2e)


def _pre_body(x_ref, tbl_ref, d_ref, pw_ref, pb_ref, h0_ref, hp_ref, den_ref):
    xb = x_ref[...]
    iota = lax.broadcasted_iota(jnp.int32, (1, H), 1)
    h = jnp.zeros((BN, H), _f32)
    for i in range(9):
        oh = (xb[:, i:i + 1] == iota).astype(_f32)
        h = h + jnp.dot(oh, tbl_ref[i], preferred_element_type=_f32)
    h0_ref[...] = h
    hp_ref[...] = jnp.dot(h, pw_ref[...], preferred_element_type=_f32) + pb_ref[...]
    den_ref[...] = d_ref[0] + d_ref[1]


def _pre_call(x, atbl, dpart, pw0, pb0):
    return pl.pallas_call(
        _pre_body,
        grid=(N // BN,),
        in_specs=[
            pl.BlockSpec((BN, 9), lambda i: (i, 0)),
            pl.BlockSpec((9, H, H), lambda i: (0, 0, 0)),
            pl.BlockSpec((2, BN, H), lambda i: (0, i, 0)),
            pl.BlockSpec((H, H), lambda i: (0, 0)),
            pl.BlockSpec((1, H), lambda i: (0, 0)),
        ],
        out_specs=[
            pl.BlockSpec((BN, H), lambda i: (i, 0)),
            pl.BlockSpec((BN, H), lambda i: (i, 0)),
            pl.BlockSpec((BN, H), lambda i: (i, 0)),
        ],
        out_shape=[
            jax.ShapeDtypeStruct((N, H), _f32),
            jax.ShapeDtypeStruct((N, H), _f32),
            jax.ShapeDtypeStruct((N, H), _f32),
        ],
    )(x, atbl, dpart, pw0, pb0)


def _tcomb_body(ec_ref, pw_ref, out_ref):
    out_ref[...] = jnp.dot(ec_ref[...], pw_ref[0],
                           preferred_element_type=_f32)[None]


def _tcomb_call(ecomb, pW):
    return pl.pallas_call(
        _tcomb_body,
        grid=(L,),
        in_specs=[
            pl.BlockSpec((H, H), lambda i: (0, 0)),
            pl.BlockSpec((1, H, H), lambda i: (i, 0, 0)),
        ],
        out_specs=pl.BlockSpec((1, H, H), lambda i: (i, 0, 0)),
        out_shape=jax.ShapeDtypeStruct((L, H, H), _f32),
    )(ecomb, pW)


def _node_mid_body(h_ref, p_ref, den_ref, w1_ref, c1_ref, w2_ref, c2_ref,
                   pwn_ref, pbn_ref, hn_ref, hpn_ref):
    den = den_ref[...]
    psum = p_ref[0] + p_ref[1]
    aggr = jnp.where(den > 0.0, psum / den, 0.0)
    h2 = h_ref[...] + aggr
    y = jnp.dot(h2, w1_ref[...], preferred_element_type=_f32) + c1_ref[...]
    y = jnp.maximum(y, 0.0)
    y = jnp.dot(y, w2_ref[...], preferred_element_type=_f32) + c2_ref[...]
    y = jnp.maximum(y, 0.0)
    hn = h2 + y
    hn_ref[...] = hn
    hpn_ref[...] = jnp.dot(hn, pwn_ref[...], preferred_element_type=_f32) + pbn_ref[...]


def _node_mid_call(h, ppart, denom, w1, c1, w2, c2, pwn, pbn):
    return pl.pallas_call(
        _node_mid_body,
        grid=(N // BN,),
        in_specs=[
            pl.BlockSpec((BN, H), lambda i: (i, 0)),
            pl.BlockSpec((2, BN, H), lambda i: (0, i, 0)),
            pl.BlockSpec((BN, H), lambda i: (i, 0)),
            pl.BlockSpec((H, H), lambda i: (0, 0)),
            pl.BlockSpec((1, H), lambda i: (0, 0)),
            pl.BlockSpec((H, H), lambda i: (0, 0)),
            pl.BlockSpec((1, H), lambda i: (0, 0)),
            pl.BlockSpec((H, H), lambda i: (0, 0)),
            pl.BlockSpec((1, H), lambda i: (0, 0)),
        ],
        out_specs=[
            pl.BlockSpec((BN, H), lambda i: (i, 0)),
            pl.BlockSpec((BN, H), lambda i: (i, 0)),
        ],
        out_shape=[
            jax.ShapeDtypeStruct((N, H), _f32),
            jax.ShapeDtypeStruct((N, H), _f32),
        ],
    )(h, ppart, denom, w1, c1, w2, c2, pwn, pbn)


def _node_last_body(h_ref, p_ref, den_ref, w1_ref, c1_ref, w2_ref, c2_ref,
                    wp_ref, bp_ref, head_ref):
    den = den_ref[...]
    psum = p_ref[0] + p_ref[1]
    aggr = jnp.where(den > 0.0, psum / den, 0.0)
    h2 = h_ref[...] + aggr
    y = jnp.dot(h2, w1_ref[...], preferred_element_type=_f32) + c1_ref[...]
    y = jnp.maximum(y, 0.0)
    y = jnp.dot(y, w2_ref[...], preferred_element_type=_f32) + c2_ref[...]
    y = jnp.maximum(y, 0.0)
    hn = h2 + y
    part = jnp.dot(jnp.sum(hn, axis=0, keepdims=True), wp_ref[...],
                   preferred_element_type=_f32)

    @pl.when(pl.program_id(0) == 0)
    def _():
        head_ref[...] = bp_ref[...]

    head_ref[...] += part


def _node_last_call(h, ppart, denom, w1, c1, w2, c2, wp, bpr):
    return pl.pallas_call(
        _node_last_body,
        grid=(N // BN,),
        in_specs=[
            pl.BlockSpec((BN, H), lambda i: (i, 0)),
            pl.BlockSpec((2, BN, H), lambda i: (0, i, 0)),
            pl.BlockSpec((BN, H), lambda i: (i, 0)),
            pl.BlockSpec((H, H), lambda i: (0, 0)),
            pl.BlockSpec((1, H), lambda i: (0, 0)),
            pl.BlockSpec((H, H), lambda i: (0, 0)),
            pl.BlockSpec((1, H), lambda i: (0, 0)),
            pl.BlockSpec((H, 1), lambda i: (0, 0)),
            pl.BlockSpec((1, 1), lambda i: (0, 0)),
        ],
        out_specs=pl.BlockSpec((1, 1), lambda i: (0, 0)),
        out_shape=jax.ShapeDtypeStruct((1, 1), _f32),
    )(h, ppart, denom, w1, c1, w2, c2, wp, bpr)


_SC_MESH = plsc.VectorSubcoreMesh(core_axis_name="c", subcore_axis_name="s")


def _sc_denom_body(expb_hbm, dst_hbm, zeros_hbm, out_hbm, acc, dst_v, expb_v):
    cid = lax.axis_index("c")
    sid = lax.axis_index("s")
    pltpu.sync_copy(zeros_hbm.at[pl.ds(sid * RPT, RPT)],
                    acc.at[pl.ds(sid * RPT, RPT)])
    plsc.subcore_barrier()
    tbase = (cid * NS + sid) * EPT

    def chunk(c, carry):
        base = tbase + c * CH
        pltpu.sync_copy(dst_hbm.at[pl.ds(base, CH)], dst_v)
        pltpu.sync_copy(expb_hbm.at[pl.ds(base, CH)], expb_v)
        pltpu.sync_copy(expb_v, acc.at[dst_v], add=True)
        return carry

    lax.fori_loop(0, NCHUNK, chunk, 0)
    plsc.subcore_barrier()
    pltpu.sync_copy(acc.at[pl.ds(sid * RPT, RPT)],
                    out_hbm.at[pl.ds(cid * NP + sid * RPT, RPT)])


def _sc_denom_call(expb, dst, zeros_n):
    return pl.kernel(
        _sc_denom_body,
        out_type=jax.ShapeDtypeStruct((NC * NP, H), _f32),
        mesh=_SC_MESH,
        scratch_types=[
            pltpu.VMEM_SHARED((NP, H), _f32),
            pltpu.VMEM((CH,), jnp.int32),
            pltpu.VMEM((CH, H), _f32),
        ],
    )(expb, dst, zeros_n)


def _gelu_mul(xv, bv):
    x3 = xv * xv * xv
    u2 = 1.5957691216057308 * (xv + 0.044715 * x3)
    ev = jnp.exp(u2)
    t = 1.0 - 2.0 / (ev + 1.0)
    return 0.5 * xv * (1.0 + t) * bv


def _sc_edge_body(hp_hbm, tc_hbm, expb_hbm, src_hbm, comb_hbm, dst_hbm,
                  zeros_hbm, out_hbm, acc, src_v, comb_v, dst_v,
                  expb_v, hp_v, ec_v, z_v, sem0, sem1):
    cid = lax.axis_index("c")
    sid = lax.axis_index("s")
    pltpu.sync_copy(zeros_hbm.at[pl.ds(sid * RPT, RPT)],
                    acc.at[pl.ds(sid * RPT, RPT)])
    plsc.subcore_barrier()
    tbase = (cid * NS + sid) * EPT

    def chunk(c, carry):
        base = tbase + c * CH
        pltpu.sync_copy(src_hbm.at[pl.ds(base, CH)], src_v)
        pltpu.sync_copy(comb_hbm.at[pl.ds(base, CH)], comb_v)
        pltpu.sync_copy(dst_hbm.at[pl.ds(base, CH)], dst_v)
        pltpu.sync_copy(expb_hbm.at[pl.ds(base, CH)], expb_v)
        g1 = pltpu.async_copy(hp_hbm.at[src_v], hp_v, sem0)
        g2 = pltpu.async_copy(tc_hbm.at[comb_v], ec_v, sem1)
        g1.wait()
        g2.wait()

        def row(r, carry2):
            for j in range(8):
                sl = pl.ds(j * 16, 16)
                z_v[r, sl] = _gelu_mul(hp_v[r, sl] + ec_v[r, sl],
                                       expb_v[r, sl])
            return carry2

        lax.fori_loop(0, CH, row, 0)
        pltpu.sync_copy(z_v, acc.at[dst_v], add=True)
        return carry

    lax.fori_loop(0, NCHUNK, chunk, 0)
    plsc.subcore_barrier()
    pltpu.sync_copy(acc.at[pl.ds(sid * RPT, RPT)],
                    out_hbm.at[pl.ds(cid * NP + sid * RPT, RPT)])


def _sc_edge_call(hp, tc, expb, src, comb, dst, zeros_n):
    return pl.kernel(
        _sc_edge_body,
        out_type=jax.ShapeDtypeStruct((NC * NP, H), _f32),
        mesh=_SC_MESH,
        scratch_types=[
            pltpu.VMEM_SHARED((NP, H), _f32),
            pltpu.VMEM((CH,), jnp.int32),
            pltpu.VMEM((CH,), jnp.int32),
            pltpu.VMEM((CH,), jnp.int32),
            pltpu.VMEM((CH, H), _f32),
            pltpu.VMEM((CH, H), _f32),
            pltpu.VMEM((CH, H), _f32),
            pltpu.VMEM((CH, H), _f32),
            pltpu.SemaphoreType.DMA,
            pltpu.SemaphoreType.DMA,
        ],
    )(hp, tc, expb, src, comb, dst, zeros_n)


def kernel(x, edge_attr, edge_index, bases, atom_emb, bond_emb, fW1, fb1,
           fs1, ft1, fW2, fb2, fs2, ft2, pW, pb, W1, b1, s1, t1, W2, b2,
           s2, t2, Wp, bp):
    src = edge_index[0]
    dst = edge_index[1]

    w1e = fW1 * fs1[None, :]
    c1e = (fb1 * fs1 + ft1)[None, :]
    w2e = fW2 * fs2[None, :]
    c2e = (fb2 * fs2 + ft2)[None, :]
    W1e = W1 * s1[:, None, :]
    C1e = (b1 * s1 + t1)[:, None, :]
    W2e = W2 * s2[:, None, :]
    C2e = (b2 * s2 + t2)[:, None, :]

    ecomb = (bond_emb[0][:, None, None, :] + bond_emb[1][None, :, None, :]
             + bond_emb[2][None, None, :, :]).reshape(125, H)
    ecomb = jnp.pad(ecomb, ((0, 3), (0, 0)))
    atbl = jnp.pad(atom_emb, ((0, 0), (0, H - 119), (0, 0)))

    zeros_n = jnp.zeros((NP, H), _f32)

    expb, comb2 = _filt_call(bases, edge_attr, w1e, c1e, w2e, c2e)
    comb = comb2.reshape(E)
    tcomb = _tcomb_call(ecomb, pW)
    dpart = _sc_denom_call(expb, dst, zeros_n).reshape(NC, NP, H)
    h, hp, denom = _pre_call(x, atbl, dpart, pW[0], pb[0][None, :])

    out = None
    for l in range(L):
        ppart = _sc_edge_call(hp, tcomb[l], expb, src, comb, dst,
                              zeros_n).reshape(NC, NP, H)
        if l < L - 1:
            h, hp = _node_mid_call(h, ppart, denom, W1e[l], C1e[l], W2e[l],
                                   C2e[l], pW[l + 1], pb[l + 1][None, :])
        else:
            out = _node_last_call(h, ppart, denom, W1e[l], C1e[l], W2e[l],
                                  C2e[l], Wp, bp[None, :])
    return out
